# Initial kernel scaffold; baseline (speedup 1.0000x reference)
import jax
import jax.numpy as jnp
from jax.experimental import pallas as pl


def kernel(x, alpha_xz, beta_xy, gamma_x, alpha_wy, beta_wx, gamma_w, alpha_xv, beta_wv, gamma_v, gate_w, gate_b, ln_w, ln_b):
    raise NotImplementedError("write your pallas kernel here")


# speedup vs baseline: 2.2405x; 2.2405x over previous
import jax
import jax.numpy as jnp
from jax.experimental import pallas as pl

D_MODEL = 1024
SEQ = 8192
BATCH = 2

_SPINE = [0, 2, 4, 12, 36, 104, 304, 888, 2592, 7568]
_WRITE_POS = _SPINE[3:]

_BLK = 256
_NBLK = SEQ // _BLK


def _recurrence_kernel(rows_ref, axz_ref, bxy_ref, gx_ref, awy_ref, bwx_ref,
                       gw_ref, axv_ref, bwv_ref, gv_ref, gwv_ref, gwz_ref,
                       gb_ref, lnw_ref, lnb_ref, out_ref):
    axz = axz_ref[...]
    bxy = bxy_ref[...]
    gx = gx_ref[...]
    awy = awy_ref[...]
    bwx = bwx_ref[...]
    gw = gw_ref[...]
    axv = axv_ref[...]
    bwv = bwv_ref[...]
    gv = gv_ref[...]
    w_v = gwv_ref[...]
    w_z = gwz_ref[...]
    gb = gb_ref[...]
    lnw = lnw_ref[...]
    lnb = lnb_ref[...]

    vals = [rows_ref[:, i, :] for i in range(len(_SPINE))]
    for k in range(3, len(_SPINE)):
        z = vals[k]
        y = vals[k - 1]
        x_prev = vals[k - 2]
        x_new = axz * z + bxy * y + gx
        w = awy * y + bwx * x_prev + gw
        v = axv * x_new + bwv * w + gv
        logits = (jnp.dot(v, w_v, preferred_element_type=jnp.float32)
                  + jnp.dot(z, w_z, preferred_element_type=jnp.float32) + gb)
        g = jax.nn.sigmoid(logits)
        gated = g * v + (1.0 - g) * z
        mean = jnp.mean(gated, axis=-1, keepdims=True)
        var = jnp.mean((gated - mean) ** 2, axis=-1, keepdims=True)
        vals[k] = (gated - mean) * jax.lax.rsqrt(var + 1e-5) * lnw + lnb
    for j, k in enumerate(range(3, len(_SPINE))):
        out_ref[:, j, :] = vals[k]


def _copy_scatter_kernel(x_ref, rows_ref, out_ref):
    out_ref[...] = x_ref[...]
    pid = pl.program_id(0)
    per_block = {}
    for j, p in enumerate(_WRITE_POS):
        per_block.setdefault(p // _BLK, []).append((p % _BLK, j))
    for b, lst in per_block.items():
        @pl.when(pid == b)
        def _():
            for off, j in lst:
                out_ref[:, off, :] = rows_ref[:, j, :]


def kernel(x, alpha_xz, beta_xy, gamma_x, alpha_wy, beta_wx, gamma_w,
           alpha_xv, beta_wv, gamma_v, gate_w, gate_b, ln_w, ln_b):
    spine_rows = x[:, jnp.array(_SPINE), :]
    w_v = gate_w[:, :D_MODEL].T
    w_z = gate_w[:, D_MODEL:].T

    new_rows = pl.pallas_call(
        _recurrence_kernel,
        out_shape=jax.ShapeDtypeStruct((BATCH, len(_WRITE_POS), D_MODEL),
                                       jnp.float32),
    )(spine_rows, alpha_xz, beta_xy, gamma_x, alpha_wy, beta_wx, gamma_w,
      alpha_xv, beta_wv, gamma_v, w_v, w_z, gate_b, ln_w, ln_b)

    out = pl.pallas_call(
        _copy_scatter_kernel,
        grid=(_NBLK,),
        in_specs=[
            pl.BlockSpec((BATCH, _BLK, D_MODEL), lambda i: (0, i, 0)),
            pl.BlockSpec((BATCH, len(_WRITE_POS), D_MODEL),
                         lambda i: (0, 0, 0)),
        ],
        out_specs=pl.BlockSpec((BATCH, _BLK, D_MODEL), lambda i: (0, i, 0)),
        out_shape=jax.ShapeDtypeStruct((BATCH, SEQ, D_MODEL), jnp.float32),
    )(x, new_rows)
    return out
